# Initial kernel scaffold; baseline (speedup 1.0000x reference)
import functools

import jax
import jax.numpy as jnp
from jax import lax
from jax.experimental import pallas as pl
from jax.experimental.pallas import tpu as pltpu
from jax.experimental.pallas import tpu_sc as plsc

N_EDGES = 6_400_000
B = 2000
NW = 32
CHUNKS_PER_W = N_EDGES // (B * NW)


def _vperm(x, idx):
    return jnp.take(x, idx, mode="promise_in_bounds")


def _sc_body(pos4_hbm, src_hbm, dst_hbm, off_hbm, crow_hbm, out_hbm,
             sidx_v, didx_v, off_v, gsrc_v, gdst_v, d2_v, dist_v, crow_v,
             sem0, sem1):
    c = lax.axis_index("c")
    s = lax.axis_index("s")
    w = s * 2 + c

    pltpu.sync_copy(crow_hbm, crow_v)
    cr0 = crow_v[pl.ds(0, 16)]
    cr1 = crow_v[pl.ds(16, 16)]
    cr2 = crow_v[pl.ds(32, 16)]

    iota = lax.iota(jnp.int32, 16)
    permx = 3 * (iota >> 2)
    perm1 = iota ^ 1
    perm2 = iota ^ 2
    mask4 = (iota & 3) == 0

    def chunk_body(j, carry):
        base = (w * CHUNKS_PER_W + j) * B
        pltpu.sync_copy(src_hbm.at[pl.ds(base, B)], sidx_v)
        pltpu.sync_copy(dst_hbm.at[pl.ds(base, B)], didx_v)
        pltpu.sync_copy(off_hbm.at[pl.ds(3 * base, 3 * B)], off_v)
        cp0 = pltpu.async_copy(pos4_hbm.at[sidx_v], gsrc_v, sem0)
        cp1 = pltpu.async_copy(pos4_hbm.at[didx_v], gdst_v, sem1)
        cp0.wait()
        cp1.wait()

        def cvec(v, carry2):
            wnd = off_v[pl.ds(12 * v, 16)]
            ox = _vperm(wnd, permx)
            oy = _vperm(wnd, permx + 1)
            oz = _vperm(wnd, permx + 2)
            pbc = ox * cr0 + oy * cr1 + oz * cr2
            gs = gsrc_v[pl.ds(16 * v, 16)]
            gd = gdst_v[pl.ds(16 * v, 16)]
            df = gd - gs - pbc
            sq = df * df
            t1 = sq + _vperm(sq, perm1)
            t2 = t1 + _vperm(t1, perm2)
            plsc.store_compressed(d2_v.at[pl.ds(4 * v, 16)], t2, mask4)
            return carry2

        lax.fori_loop(0, B // 4, cvec, 0)

        def svec(u, carry2):
            x = d2_v[pl.ds(16 * u, 16)]
            xc = jnp.maximum(x, jnp.float32(1e-30))
            i = lax.bitcast_convert_type(xc, jnp.int32)
            y = lax.bitcast_convert_type(0x5F3759DF - (i >> 1), jnp.float32)
            for _ in range(3):
                y = y * (1.5 - 0.5 * xc * y * y)
            dist_v[pl.ds(16 * u, 16)] = x * y
            return carry2

        lax.fori_loop(0, B // 16, svec, 0)
        pltpu.sync_copy(dist_v, out_hbm.at[pl.ds(base, B)])
        return carry

    lax.fori_loop(0, CHUNKS_PER_W, chunk_body, 0)


@jax.jit
def kernel(pos, edge_index, offsets, cell_vectors):
    pos4 = jnp.pad(pos, ((0, 0), (0, 1)))
    src = edge_index[0].astype(jnp.int32)
    dst = edge_index[1].astype(jnp.int32)
    offf = offsets.reshape(-1)
    crow = jnp.tile(jnp.pad(cell_vectors.astype(jnp.float32),
                            ((0, 0), (0, 1))), (1, 4)).reshape(-1)

    mesh = plsc.VectorSubcoreMesh(core_axis_name="c", subcore_axis_name="s")
    f = pl.kernel(
        _sc_body,
        mesh=mesh,
        out_type=jax.ShapeDtypeStruct((N_EDGES,), jnp.float32),
        scratch_types=[
            pltpu.VMEM((B,), jnp.int32),
            pltpu.VMEM((B,), jnp.int32),
            pltpu.VMEM((3 * B,), jnp.float32),
            pltpu.VMEM((B, 4), jnp.float32),
            pltpu.VMEM((B, 4), jnp.float32),
            pltpu.VMEM((B + 16,), jnp.float32),
            pltpu.VMEM((B,), jnp.float32),
            pltpu.VMEM((48,), jnp.float32),
            pltpu.SemaphoreType.DMA,
            pltpu.SemaphoreType.DMA,
        ],
    )
    return f(pos4, src, dst, offf, crow)


# speedup vs baseline: 5.1511x; 5.1511x over previous
import jax
import jax.numpy as jnp
from jax import lax
from jax.experimental import pallas as pl
from jax.experimental.pallas import tpu as pltpu
from jax.experimental.pallas import tpu_sc as plsc

N_NODES_ = 100_000
N_EDGES_ = 6_400_000
B = 2000
NW = 32
CHUNKS_PER_W = N_EDGES_ // (B * NW)

_GATHER_DNUMS = lax.GatherDimensionNumbers(
    offset_dims=(), collapsed_slice_dims=(0,), start_index_map=(0,))


def _vperm(x, idx):
    return lax.gather(x, idx[:, None], _GATHER_DNUMS, slice_sizes=(1,),
                      mode=lax.GatherScatterMode.PROMISE_IN_BOUNDS)


def _sc_body(px_hbm, py_hbm, pz_hbm, src_hbm, dst_hbm, off_hbm, cell_hbm,
             out_hbm,
             px_s, py_s, pz_s,
             sidx_v, didx_v, off_v,
             gxs_v, gys_v, gzs_v, gxd_v, gyd_v, gzd_v,
             dist_v, cell_v, semg):
    c = lax.axis_index("c")
    s = lax.axis_index("s")
    w = s * 2 + c

    @pl.when(s == 0)
    def _fill():
        pltpu.sync_copy(px_hbm, px_s)
        pltpu.sync_copy(py_hbm, py_s)
        pltpu.sync_copy(pz_hbm, pz_s)

    pltpu.sync_copy(cell_hbm, cell_v)
    plsc.subcore_barrier()

    iota = lax.iota(jnp.int32, 16)
    zero = iota * 0
    cvec = cell_v[pl.ds(0, 16)]
    cw = [[_vperm(cvec, zero + (3 * r + k)) for k in range(3)]
          for r in range(3)]
    flat = [3 * iota + cc for cc in range(3)]
    loc = [f & 15 for f in flat]
    in0 = [(f >> 4) == 0 for f in flat]
    in1 = [(f >> 4) == 1 for f in flat]

    def chunk_body(j, carry):
        base = (w * CHUNKS_PER_W + j) * B
        pltpu.sync_copy(src_hbm.at[pl.ds(base, B)], sidx_v)
        pltpu.sync_copy(dst_hbm.at[pl.ds(base, B)], didx_v)
        pltpu.sync_copy(off_hbm.at[pl.ds(3 * base, 3 * B)], off_v)
        cps = [
            pltpu.async_copy(px_s.at[sidx_v], gxs_v, semg),
            pltpu.async_copy(py_s.at[sidx_v], gys_v, semg),
            pltpu.async_copy(pz_s.at[sidx_v], gzs_v, semg),
            pltpu.async_copy(px_s.at[didx_v], gxd_v, semg),
            pltpu.async_copy(py_s.at[didx_v], gyd_v, semg),
            pltpu.async_copy(pz_s.at[didx_v], gzd_v, semg),
        ]
        for cp in cps:
            cp.wait()

        def cvec(v, carry2):
            w0 = off_v[pl.ds(48 * v, 16)]
            w1 = off_v[pl.ds(48 * v + 16, 16)]
            w2 = off_v[pl.ds(48 * v + 32, 16)]
            o = [jnp.where(in0[cc], _vperm(w0, loc[cc]),
                           jnp.where(in1[cc], _vperm(w1, loc[cc]),
                                     _vperm(w2, loc[cc])))
                 for cc in range(3)]
            pbx = o[0] * cw[0][0] + o[1] * cw[1][0] + o[2] * cw[2][0]
            pby = o[0] * cw[0][1] + o[1] * cw[1][1] + o[2] * cw[2][1]
            pbz = o[0] * cw[0][2] + o[1] * cw[1][2] + o[2] * cw[2][2]
            sl = pl.ds(16 * v, 16)
            dx = gxd_v[sl] - gxs_v[sl] - pbx
            dy = gyd_v[sl] - gys_v[sl] - pby
            dz = gzd_v[sl] - gzs_v[sl] - pbz
            d2 = dx * dx + dy * dy + dz * dz
            xc = jnp.maximum(d2, jnp.float32(1e-30))
            i = lax.bitcast_convert_type(xc, jnp.int32)
            y = lax.bitcast_convert_type(0x5F3759DF - (i >> 1), jnp.float32)
            for _ in range(3):
                y = y * (1.5 - 0.5 * xc * y * y)
            dist_v[sl] = d2 * y
            return carry2

        lax.fori_loop(0, B // 16, cvec, 0)
        pltpu.sync_copy(dist_v, out_hbm.at[pl.ds(base, B)])
        return carry

    lax.fori_loop(0, CHUNKS_PER_W, chunk_body, 0)


@jax.jit
def kernel(pos, edge_index, offsets, cell_vectors):
    posx = jnp.asarray(pos[:, 0], jnp.float32)
    posy = jnp.asarray(pos[:, 1], jnp.float32)
    posz = jnp.asarray(pos[:, 2], jnp.float32)
    src = edge_index[0].astype(jnp.int32)
    dst = edge_index[1].astype(jnp.int32)
    offf = offsets.reshape(-1)
    cellf = jnp.pad(cell_vectors.astype(jnp.float32).reshape(-1), (0, 7))

    mesh = plsc.VectorSubcoreMesh(core_axis_name="c", subcore_axis_name="s")
    f = pl.kernel(
        _sc_body,
        mesh=mesh,
        out_type=jax.ShapeDtypeStruct((N_EDGES_,), jnp.float32),
        scratch_types=[
            pltpu.VMEM_SHARED((N_NODES_,), jnp.float32),
            pltpu.VMEM_SHARED((N_NODES_,), jnp.float32),
            pltpu.VMEM_SHARED((N_NODES_,), jnp.float32),
            pltpu.VMEM((B,), jnp.int32),
            pltpu.VMEM((B,), jnp.int32),
            pltpu.VMEM((3 * B,), jnp.float32),
            pltpu.VMEM((B,), jnp.float32),
            pltpu.VMEM((B,), jnp.float32),
            pltpu.VMEM((B,), jnp.float32),
            pltpu.VMEM((B,), jnp.float32),
            pltpu.VMEM((B,), jnp.float32),
            pltpu.VMEM((B,), jnp.float32),
            pltpu.VMEM((B,), jnp.float32),
            pltpu.VMEM((16,), jnp.float32),
            pltpu.SemaphoreType.DMA,
        ],
    )
    return f(posx, posy, posz, src, dst, offf, cellf)
